# baseline (device time: 44586 ns/iter reference)
import jax
import jax.numpy as jnp
from jax import lax
from jax.experimental import pallas as pl
from jax.experimental.pallas import tpu as pltpu

N_DEV = 8
E_PER = 4
R = 192


def kernel(x, router_W, route_idx, expert_W):
    n, d = x.shape
    e_loc, _, h = expert_W.shape

    def body(x_ref, idx_ref, w_ref, out_ref,
             wbf_ref, send_ref, recv_ref, send_sems, recv_sems):
        me = lax.axis_index("i")
        f32 = jnp.float32
        bf16 = jnp.bfloat16

        barrier_sem = pltpu.get_barrier_semaphore()
        for j in range(1, N_DEV):
            peer = lax.rem(me + j, N_DEV)
            pl.semaphore_signal(
                barrier_sem, inc=1,
                device_id=(peer,), device_id_type=pl.DeviceIdType.MESH,
            )
        pl.semaphore_wait(barrier_sem, N_DEV - 1)

        wbf_ref[...] = w_ref[...].astype(bf16)

        idx = idx_ref[...]
        owner = idx // E_PER
        row_i = lax.broadcasted_iota(jnp.int32, (n, n), 0)
        col_i = lax.broadcasted_iota(jnp.int32, (n, n), 1)
        T = (row_i > col_i).astype(bf16)

        rr = lax.broadcasted_iota(jnp.int32, (1, R), 1).astype(f32)

        def gather_onehot(s):
            mask_s = (owner == s).astype(bf16)
            rank_s = jnp.dot(T, mask_s, preferred_element_type=f32)
            return ((rank_s == rr).astype(bf16) * mask_s)

        gt_me = gather_onehot(me)
        xb = x_ref[...].astype(bf16)
        cdims = (((0,), (0,)), ((), ()))
        cx = lax.dot_general(gt_me, xb, cdims,
                             preferred_element_type=f32).astype(bf16)
        ce = lax.dot_general(gt_me, idx.astype(bf16), cdims,
                             preferred_element_type=f32)
        acc = jnp.zeros((R, h), f32)
        for k in range(E_PER):
            eid = (me * E_PER + k).astype(f32)
            xk = jnp.where(ce == eid, cx, jnp.zeros_like(cx))
            acc = acc + jnp.dot(xk, wbf_ref[k], preferred_element_type=f32)
        send_ref[...] = acc.astype(bf16)

        sends = []
        for j in range(1, N_DEV):
            p = lax.rem(me + j, N_DEV)
            rdma = pltpu.make_async_remote_copy(
                src_ref=send_ref,
                dst_ref=recv_ref.at[N_DEV - j],
                send_sem=send_sems.at[j],
                recv_sem=recv_sems.at[N_DEV - j],
                device_id=(p,),
                device_id_type=pl.DeviceIdType.MESH,
            )
            rdma.start()
            sends.append(rdma)

        out_ref[...] = jnp.dot(gt_me, send_ref[...],
                               preferred_element_type=f32)

        for k in range(N_DEV - 1, 0, -1):
            sends[(N_DEV - k) - 1].wait_recv()
            s = lax.rem(me + k, N_DEV)
            gt_s = gather_onehot(s)
            out_ref[...] += jnp.dot(gt_s, recv_ref[k],
                                    preferred_element_type=f32)

        for rdma in sends:
            rdma.wait_send()

    return pl.pallas_call(
        body,
        out_shape=jax.ShapeDtypeStruct((n, h), jnp.float32),
        in_specs=[
            pl.BlockSpec(memory_space=pltpu.VMEM),
            pl.BlockSpec(memory_space=pltpu.VMEM),
            pl.BlockSpec(memory_space=pltpu.VMEM),
        ],
        out_specs=pl.BlockSpec(memory_space=pltpu.VMEM),
        scratch_shapes=[
            pltpu.VMEM((e_loc, d, h), jnp.bfloat16),
            pltpu.VMEM((R, h), jnp.bfloat16),
            pltpu.VMEM((N_DEV, R, h), jnp.bfloat16),
            pltpu.SemaphoreType.DMA((N_DEV,)),
            pltpu.SemaphoreType.DMA((N_DEV,)),
        ],
        compiler_params=pltpu.CompilerParams(collective_id=0),
    )(x, route_idx, expert_W)
